# baseline (speedup 1.0000x reference)
import functools

import jax
import jax.numpy as jnp
from jax import lax
from jax.experimental import pallas as pl
from jax.experimental.pallas import tpu as pltpu
from jax.experimental.pallas import tpu_sc as plsc

N_TOK = 4096
D_MODEL = 1024
NUM_SPEC = 8
NUM_SHARED = 2
NUM_TOTAL = NUM_SPEC + NUM_SHARED
BT = 2048

_NW = 32
_TPW = N_TOK // _NW
_NCHUNK = _TPW // 16


def _sc_routing_body(slT_hbm, shlT_hbm, ct_hbm, sl_v, shl_v, ct_v):
    wid = lax.axis_index("c") * 16 + lax.axis_index("s")
    base = wid * _TPW
    for e in range(NUM_SPEC):
        pltpu.sync_copy(slT_hbm.at[e, pl.ds(base, _TPW)], sl_v.at[e])
    for e in range(NUM_SHARED):
        pltpu.sync_copy(shlT_hbm.at[e, pl.ds(base, _TPW)], shl_v.at[e])
    zeros = jnp.zeros((16,), jnp.float32)
    for i in range(_NCHUNK):
        sl = [sl_v[e, pl.ds(i * 16, 16)] for e in range(NUM_SPEC)]
        m = sl[0]
        for e in range(1, NUM_SPEC):
            m = jnp.maximum(m, sl[e])
        ex = [jnp.exp(sl[e] - m) for e in range(NUM_SPEC)]
        z = ex[0]
        for e in range(1, NUM_SPEC):
            z = z + ex[e]
        g1 = ex[0]
        for e in range(1, NUM_SPEC):
            g1 = jnp.maximum(g1, ex[e])
        a1 = jnp.full((16,), NUM_SPEC, jnp.int32)
        for e in range(NUM_SPEC - 1, -1, -1):
            a1 = jnp.where(ex[e] == g1, e, a1)
        ex2 = [jnp.where(a1 == e, 0.0, ex[e]) for e in range(NUM_SPEC)]
        g2 = ex2[0]
        for e in range(1, NUM_SPEC):
            g2 = jnp.maximum(g2, ex2[e])
        a2 = jnp.full((16,), NUM_SPEC, jnp.int32)
        for e in range(NUM_SPEC - 1, -1, -1):
            a2 = jnp.where(ex2[e] == g2, e, a2)
        denom = g1 + g2 + 1e-6 * z
        w1 = g1 / denom
        w2 = g2 / denom
        for e in range(NUM_SPEC):
            ct_v[e, pl.ds(i * 16, 16)] = jnp.where(
                a1 == e, w1, jnp.where(a2 == e, w2, zeros))
        s0 = shl_v[0, pl.ds(i * 16, 16)]
        s1 = shl_v[1, pl.ds(i * 16, 16)]
        sm = jnp.maximum(s0, s1)
        se0 = jnp.exp(s0 - sm)
        se1 = jnp.exp(s1 - sm)
        ssum = se0 + se1
        ct_v[NUM_SPEC, pl.ds(i * 16, 16)] = se0 / ssum
        ct_v[NUM_SPEC + 1, pl.ds(i * 16, 16)] = se1 / ssum
        for e in range(NUM_SPEC + NUM_SHARED, 16):
            ct_v[e, pl.ds(i * 16, 16)] = zeros
    for e in range(16):
        pltpu.sync_copy(ct_v.at[e], ct_hbm.at[e, pl.ds(base, _TPW)])


def _sc_routing(slT, shlT):
    mesh = plsc.VectorSubcoreMesh(core_axis_name="c", subcore_axis_name="s")
    kfn = functools.partial(
        pl.kernel,
        out_type=jax.ShapeDtypeStruct((16, N_TOK), jnp.float32),
        mesh=mesh,
        scratch_types=[
            pltpu.VMEM((NUM_SPEC, _TPW), jnp.float32),
            pltpu.VMEM((NUM_SHARED, _TPW), jnp.float32),
            pltpu.VMEM((16, _TPW), jnp.float32),
        ],
    )(_sc_routing_body)
    return kfn(slT, shlT)


def _moe_body(ct_in_ref, x_ref, wspec_ref, wshared_ref, o_ref):
    t = pl.program_id(0)
    e = pl.program_id(1)

    onehot_rows = (jax.lax.broadcasted_iota(jnp.int32, (16, D_MODEL), 0) == e).astype(jnp.bfloat16)
    scale_bc = jax.lax.dot_general(
        ct_in_ref[:, pl.ds(t * BT, BT)].astype(jnp.bfloat16), onehot_rows,
        (((0,), (0,)), ((), ())),
        preferred_element_type=jnp.float32)
    xs = scale_bc.astype(jnp.bfloat16) * x_ref[...]

    def _acc(w_ref):
        contrib = jnp.dot(xs, w_ref[0].astype(jnp.bfloat16), preferred_element_type=jnp.float32)

        @pl.when(e == 0)
        def _init():
            o_ref[...] = contrib

        @pl.when(e > 0)
        def _add():
            o_ref[...] += contrib

    @pl.when(e < NUM_SPEC)
    def _spec():
        _acc(wspec_ref)

    @pl.when(e >= NUM_SPEC)
    def _shared():
        _acc(wshared_ref)


def kernel(x, spec_router_logits, shared_router_logits, spec_expert_weights, shared_expert_weights):
    x16 = x.astype(jnp.bfloat16)
    ct = _sc_routing(spec_router_logits.T, shared_router_logits.T)
    return pl.pallas_call(
        _moe_body,
        grid=(N_TOK // BT, NUM_TOTAL),
        in_specs=[
            pl.BlockSpec((16, N_TOK), lambda t, e: (0, 0)),
            pl.BlockSpec((BT, D_MODEL), lambda t, e: (t, 0)),
            pl.BlockSpec((1, D_MODEL, D_MODEL), lambda t, e: (jnp.minimum(e, NUM_SPEC - 1), 0, 0)),
            pl.BlockSpec((1, D_MODEL, D_MODEL), lambda t, e: (jnp.maximum(e - NUM_SPEC, 0), 0, 0)),
        ],
        out_specs=pl.BlockSpec((BT, D_MODEL), lambda t, e: (t, 0)),
        out_shape=jax.ShapeDtypeStruct((N_TOK, D_MODEL), jnp.float32),
        compiler_params=pltpu.CompilerParams(
            dimension_semantics=("arbitrary", "arbitrary"),
        ),
    )(ct, x16, spec_expert_weights, shared_expert_weights)


# speedup vs baseline: 1.1456x; 1.1456x over previous
import jax
import jax.numpy as jnp
from jax.experimental import pallas as pl
from jax.experimental.pallas import tpu as pltpu

N_TOK = 4096
D_MODEL = 1024
NUM_SPEC = 8
NUM_SHARED = 2
NUM_TOTAL = NUM_SPEC + NUM_SHARED
BT = 2048


def _routing(slT, shlT):
    m = jnp.max(slT, axis=0, keepdims=True)
    ex = jnp.exp(slT - m)
    z = jnp.sum(ex, axis=0, keepdims=True)
    sub8 = jax.lax.broadcasted_iota(jnp.int32, slT.shape, 0)
    g1 = jnp.max(ex, axis=0, keepdims=True)
    a1 = jnp.min(jnp.where(ex == g1, sub8, NUM_SPEC), axis=0, keepdims=True)
    ex2 = jnp.where(sub8 == a1, 0.0, ex)
    g2 = jnp.max(ex2, axis=0, keepdims=True)
    a2 = jnp.min(jnp.where(ex2 == g2, sub8, NUM_SPEC), axis=0, keepdims=True)
    denom = g1 + g2 + 1e-6 * z
    w1 = g1 / denom
    w2 = g2 / denom
    sub16 = jax.lax.broadcasted_iota(jnp.int32, (16, slT.shape[1]), 0)
    ct = w1 * (sub16 == a1).astype(jnp.float32) + w2 * (sub16 == a2).astype(jnp.float32)
    sm = jnp.max(shlT, axis=0, keepdims=True)
    sex = jnp.exp(shlT - sm)
    ssum = jnp.sum(sex, axis=0, keepdims=True)
    ct = ct + (sex[0:1, :] / ssum) * (sub16 == NUM_SPEC).astype(jnp.float32)
    ct = ct + (sex[1:2, :] / ssum) * (sub16 == NUM_SPEC + 1).astype(jnp.float32)
    return ct


def _moe_body(slT_ref, shlT_ref, x_ref, wspec_ref, wshared_ref, o_ref, ct_ref):
    t = pl.program_id(0)
    e = pl.program_id(1)

    @pl.when((t == 0) & (e == 0))
    def _do_routing():
        ct_ref[...] = _routing(slT_ref[...], shlT_ref[...]).astype(jnp.bfloat16)

    onehot_rows = (jax.lax.broadcasted_iota(jnp.int32, (16, 128), 0) == e).astype(jnp.bfloat16)
    scale_nar = jax.lax.dot_general(
        ct_ref[:, pl.ds(t * BT, BT)], onehot_rows, (((0,), (0,)), ((), ())),
        preferred_element_type=jnp.float32)
    s16 = scale_nar.astype(jnp.bfloat16)
    scale_bc = jnp.concatenate([s16] * (D_MODEL // 128), axis=1)
    xs = scale_bc * x_ref[...]

    def _acc(w_ref):
        contrib = jnp.dot(xs, w_ref[0].astype(jnp.bfloat16), preferred_element_type=jnp.float32)

        @pl.when(e == 0)
        def _init():
            o_ref[...] = contrib

        @pl.when(e > 0)
        def _add():
            o_ref[...] += contrib

    @pl.when(e < NUM_SPEC)
    def _spec():
        _acc(wspec_ref)

    @pl.when(e >= NUM_SPEC)
    def _shared():
        _acc(wshared_ref)


def kernel(x, spec_router_logits, shared_router_logits, spec_expert_weights, shared_expert_weights):
    x16 = x.astype(jnp.bfloat16)
    return pl.pallas_call(
        _moe_body,
        grid=(N_TOK // BT, NUM_TOTAL),
        in_specs=[
            pl.BlockSpec((NUM_SPEC, N_TOK), lambda t, e: (0, 0)),
            pl.BlockSpec((NUM_SHARED, N_TOK), lambda t, e: (0, 0)),
            pl.BlockSpec((BT, D_MODEL), lambda t, e: (t, 0)),
            pl.BlockSpec((1, D_MODEL, D_MODEL), lambda t, e: (jnp.minimum(e, NUM_SPEC - 1), 0, 0)),
            pl.BlockSpec((1, D_MODEL, D_MODEL), lambda t, e: (jnp.maximum(e - NUM_SPEC, 0), 0, 0)),
        ],
        out_specs=pl.BlockSpec((BT, D_MODEL), lambda t, e: (t, 0)),
        out_shape=jax.ShapeDtypeStruct((N_TOK, D_MODEL), jnp.float32),
        scratch_shapes=[pltpu.VMEM((16, N_TOK), jnp.bfloat16)],
        compiler_params=pltpu.CompilerParams(
            dimension_semantics=("arbitrary", "arbitrary"),
        ),
    )(spec_router_logits.T, shared_router_logits.T, x16, spec_expert_weights, shared_expert_weights)


# speedup vs baseline: 1.2351x; 1.0782x over previous
import jax
import jax.numpy as jnp
from jax.experimental import pallas as pl
from jax.experimental.pallas import tpu as pltpu

N_TOK = 4096
D_MODEL = 1024
NUM_SPEC = 8
NUM_SHARED = 2
NUM_TOTAL = NUM_SPEC + NUM_SHARED
BT = 1024


def _routing(slT, shlT):
    m = jnp.max(slT, axis=0, keepdims=True)
    ex = jnp.exp(slT - m)
    z = jnp.sum(ex, axis=0, keepdims=True)
    sub8 = jax.lax.broadcasted_iota(jnp.int32, slT.shape, 0)
    g1 = jnp.max(ex, axis=0, keepdims=True)
    a1 = jnp.min(jnp.where(ex == g1, sub8, NUM_SPEC), axis=0, keepdims=True)
    ex2 = jnp.where(sub8 == a1, 0.0, ex)
    g2 = jnp.max(ex2, axis=0, keepdims=True)
    a2 = jnp.min(jnp.where(ex2 == g2, sub8, NUM_SPEC), axis=0, keepdims=True)
    denom = g1 + g2 + 1e-6 * z
    w1 = g1 / denom
    w2 = g2 / denom
    sub16 = jax.lax.broadcasted_iota(jnp.int32, (16, slT.shape[1]), 0)
    ct = w1 * (sub16 == a1).astype(jnp.float32) + w2 * (sub16 == a2).astype(jnp.float32)
    sm = jnp.max(shlT, axis=0, keepdims=True)
    sex = jnp.exp(shlT - sm)
    ssum = jnp.sum(sex, axis=0, keepdims=True)
    ct = ct + (sex[0:1, :] / ssum) * (sub16 == NUM_SPEC).astype(jnp.float32)
    ct = ct + (sex[1:2, :] / ssum) * (sub16 == NUM_SPEC + 1).astype(jnp.float32)
    return ct


def _moe_body(slT_ref, shlT_ref, x_ref, wspec_ref, wshared_ref, o_ref, ct_ref):
    t = pl.program_id(0)
    p = pl.program_id(1)

    @pl.when((t == 0) & (p == 0))
    def _do_routing():
        ct_ref[...] = _routing(slT_ref[...], shlT_ref[...]).astype(jnp.bfloat16)

    lane256 = jax.lax.broadcasted_iota(jnp.int32, (16, 256), 1)
    sub16 = jax.lax.broadcasted_iota(jnp.int32, (16, 256), 0)
    target = jnp.where(lane256 < 128, 2 * p, 2 * p + 1)
    onehot2 = (sub16 == target).astype(jnp.bfloat16)
    scale_nar = jax.lax.dot_general(
        ct_ref[:, pl.ds(t * BT, BT)], onehot2, (((0,), (0,)), ((), ())),
        preferred_element_type=jnp.float32)
    s16 = scale_nar.astype(jnp.bfloat16)
    rep = D_MODEL // 128
    bcA = jnp.concatenate([s16[:, 0:128]] * rep, axis=1)
    bcB = jnp.concatenate([s16[:, 128:256]] * rep, axis=1)
    x_blk = x_ref[...]
    xs2 = jnp.concatenate([bcA * x_blk, bcB * x_blk], axis=1)

    def _acc(w_ref):
        contrib = jnp.dot(xs2, w_ref[0].astype(jnp.bfloat16), preferred_element_type=jnp.float32)

        @pl.when(p == 0)
        def _init():
            o_ref[...] = contrib

        @pl.when(p > 0)
        def _add():
            o_ref[...] += contrib

    @pl.when(p < NUM_SPEC // 2)
    def _spec():
        _acc(wspec_ref)

    @pl.when(p >= NUM_SPEC // 2)
    def _shared():
        _acc(wshared_ref)


def kernel(x, spec_router_logits, shared_router_logits, spec_expert_weights, shared_expert_weights):
    x16 = x.astype(jnp.bfloat16)
    wspec4 = spec_expert_weights.reshape(NUM_SPEC // 2, 2 * D_MODEL, D_MODEL)
    wshared1 = shared_expert_weights.reshape(NUM_SHARED // 2, 2 * D_MODEL, D_MODEL)
    npair = NUM_TOTAL // 2
    return pl.pallas_call(
        _moe_body,
        grid=(N_TOK // BT, npair),
        in_specs=[
            pl.BlockSpec((NUM_SPEC, N_TOK), lambda t, p: (0, 0)),
            pl.BlockSpec((NUM_SHARED, N_TOK), lambda t, p: (0, 0)),
            pl.BlockSpec((BT, D_MODEL), lambda t, p: (t, 0)),
            pl.BlockSpec((1, 2 * D_MODEL, D_MODEL), lambda t, p: (jnp.minimum(p, NUM_SPEC // 2 - 1), 0, 0)),
            pl.BlockSpec((1, 2 * D_MODEL, D_MODEL), lambda t, p: (0, 0, 0)),
        ],
        out_specs=pl.BlockSpec((BT, D_MODEL), lambda t, p: (t, 0)),
        out_shape=jax.ShapeDtypeStruct((N_TOK, D_MODEL), jnp.float32),
        scratch_shapes=[pltpu.VMEM((16, N_TOK), jnp.bfloat16)],
        compiler_params=pltpu.CompilerParams(
            dimension_semantics=("arbitrary", "arbitrary"),
        ),
    )(spec_router_logits.T, shared_router_logits.T, x16, wspec4, wshared1)


# speedup vs baseline: 1.2412x; 1.0049x over previous
import jax
import jax.numpy as jnp
from jax.experimental import pallas as pl
from jax.experimental.pallas import tpu as pltpu

N_TOK = 4096
D_MODEL = 1024
NUM_SPEC = 8
NUM_SHARED = 2
NUM_TOTAL = NUM_SPEC + NUM_SHARED
BT = 1024


def _routing(slT, shlT):
    m = jnp.max(slT, axis=0, keepdims=True)
    ex = jnp.exp(slT - m)
    z = jnp.sum(ex, axis=0, keepdims=True)
    sub8 = jax.lax.broadcasted_iota(jnp.int32, slT.shape, 0)
    g1 = jnp.max(ex, axis=0, keepdims=True)
    a1 = jnp.min(jnp.where(ex == g1, sub8, NUM_SPEC), axis=0, keepdims=True)
    ex2 = jnp.where(sub8 == a1, 0.0, ex)
    g2 = jnp.max(ex2, axis=0, keepdims=True)
    a2 = jnp.min(jnp.where(ex2 == g2, sub8, NUM_SPEC), axis=0, keepdims=True)
    denom = g1 + g2 + 1e-6 * z
    w1 = g1 / denom
    w2 = g2 / denom
    sub16 = jax.lax.broadcasted_iota(jnp.int32, (16, slT.shape[1]), 0)
    ct = w1 * (sub16 == a1).astype(jnp.float32) + w2 * (sub16 == a2).astype(jnp.float32)
    sm = jnp.max(shlT, axis=0, keepdims=True)
    sex = jnp.exp(shlT - sm)
    ssum = jnp.sum(sex, axis=0, keepdims=True)
    ct = ct + (sex[0:1, :] / ssum) * (sub16 == NUM_SPEC).astype(jnp.float32)
    ct = ct + (sex[1:2, :] / ssum) * (sub16 == NUM_SPEC + 1).astype(jnp.float32)
    return ct


def _moe_body(slT_ref, shlT_ref, x_ref, wspec_ref, wshared_ref, o_ref, ct_ref):
    t = pl.program_id(0)
    p = pl.program_id(1)

    @pl.when((t == 0) & (p == 0))
    def _do_routing():
        ct_ref[...] = _routing(slT_ref[...], shlT_ref[...]).astype(jnp.bfloat16)

    lane256 = jax.lax.broadcasted_iota(jnp.int32, (16, 256), 1)
    sub16 = jax.lax.broadcasted_iota(jnp.int32, (16, 256), 0)
    target = jnp.where(lane256 < 128, 2 * p, 2 * p + 1)
    onehot2 = (sub16 == target).astype(jnp.bfloat16)
    scale_nar = jax.lax.dot_general(
        ct_ref[:, pl.ds(t * BT, BT)], onehot2, (((0,), (0,)), ((), ())),
        preferred_element_type=jnp.float32)
    s16 = scale_nar.astype(jnp.bfloat16)
    rep = D_MODEL // 128
    bcA = jnp.concatenate([s16[:, 0:128]] * rep, axis=1)
    bcB = jnp.concatenate([s16[:, 128:256]] * rep, axis=1)
    x_blk = x_ref[...]
    xs2 = jnp.concatenate([bcA * x_blk, bcB * x_blk], axis=1)

    def _acc(w_ref):
        contrib = jnp.dot(xs2, w_ref[0].astype(jnp.bfloat16), preferred_element_type=jnp.float32)

        @pl.when(p == 0)
        def _init():
            o_ref[...] = contrib

        @pl.when(p > 0)
        def _add():
            o_ref[...] += contrib

    @pl.when(p < NUM_SPEC // 2)
    def _spec():
        _acc(wspec_ref)

    @pl.when(p >= NUM_SPEC // 2)
    def _shared():
        _acc(wshared_ref)


def kernel(x, spec_router_logits, shared_router_logits, spec_expert_weights, shared_expert_weights):
    x16 = x.astype(jnp.bfloat16)
    wspec4 = spec_expert_weights.reshape(NUM_SPEC // 2, 2 * D_MODEL, D_MODEL)
    wshared1 = shared_expert_weights.reshape(NUM_SHARED // 2, 2 * D_MODEL, D_MODEL)
    npair = NUM_TOTAL // 2
    return pl.pallas_call(
        _moe_body,
        grid=(N_TOK // BT, npair),
        in_specs=[
            pl.BlockSpec((NUM_SPEC, N_TOK), lambda t, p: (0, 0)),
            pl.BlockSpec((NUM_SHARED, N_TOK), lambda t, p: (0, 0)),
            pl.BlockSpec((BT, D_MODEL), lambda t, p: (t, 0)),
            pl.BlockSpec((1, 2 * D_MODEL, D_MODEL), lambda t, p: (jnp.minimum(p, NUM_SPEC // 2 - 1), 0, 0)),
            pl.BlockSpec((1, 2 * D_MODEL, D_MODEL), lambda t, p: (0, 0, 0)),
        ],
        out_specs=pl.BlockSpec((BT, D_MODEL), lambda t, p: (t, 0)),
        out_shape=jax.ShapeDtypeStruct((N_TOK, D_MODEL), jnp.float32),
        scratch_shapes=[pltpu.VMEM((16, N_TOK), jnp.bfloat16)],
        compiler_params=pltpu.CompilerParams(
            dimension_semantics=("parallel", "arbitrary"),
        ),
    )(spec_router_logits.T, shared_router_logits.T, x16, wspec4, wshared1)


# speedup vs baseline: 1.2498x; 1.0069x over previous
import jax
import jax.numpy as jnp
from jax.experimental import pallas as pl
from jax.experimental.pallas import tpu as pltpu

N_TOK = 4096
D_MODEL = 1024
NUM_SPEC = 8
NUM_SHARED = 2
NUM_TOTAL = NUM_SPEC + NUM_SHARED
BT = 1024


def _routing(slT, shlT):
    m = jnp.max(slT, axis=0, keepdims=True)
    ex = jnp.exp(slT - m)
    z = jnp.sum(ex, axis=0, keepdims=True)
    sub8 = jax.lax.broadcasted_iota(jnp.int32, slT.shape, 0)
    g1 = jnp.max(ex, axis=0, keepdims=True)
    a1 = jnp.min(jnp.where(ex == g1, sub8, NUM_SPEC), axis=0, keepdims=True)
    ex2 = jnp.where(sub8 == a1, 0.0, ex)
    g2 = jnp.max(ex2, axis=0, keepdims=True)
    a2 = jnp.min(jnp.where(ex2 == g2, sub8, NUM_SPEC), axis=0, keepdims=True)
    denom = g1 + g2 + 1e-6 * z
    w1 = g1 / denom
    w2 = g2 / denom
    sub16 = jax.lax.broadcasted_iota(jnp.int32, (16, slT.shape[1]), 0)
    ct = w1 * (sub16 == a1).astype(jnp.float32) + w2 * (sub16 == a2).astype(jnp.float32)
    sm = jnp.max(shlT, axis=0, keepdims=True)
    sex = jnp.exp(shlT - sm)
    ssum = jnp.sum(sex, axis=0, keepdims=True)
    ct = ct + (sex[0:1, :] / ssum) * (sub16 == NUM_SPEC).astype(jnp.float32)
    ct = ct + (sex[1:2, :] / ssum) * (sub16 == NUM_SPEC + 1).astype(jnp.float32)
    return ct


def _moe_body(slT_ref, shlT_ref, x_ref, wspec_ref, wshared_ref, o_ref, ct_ref, wb_ref):
    p = pl.program_id(0)
    t = pl.program_id(1)

    @pl.when((t == 0) & (p == 0))
    def _do_routing():
        ct_ref[...] = _routing(slT_ref[...], shlT_ref[...]).astype(jnp.bfloat16)

    @pl.when(t == 0)
    def _cvt_w():
        @pl.when(p < NUM_SPEC // 2)
        def _s():
            wb_ref[...] = wspec_ref[0].astype(jnp.bfloat16)

        @pl.when(p >= NUM_SPEC // 2)
        def _h():
            wb_ref[...] = wshared_ref[0].astype(jnp.bfloat16)

    lane256 = jax.lax.broadcasted_iota(jnp.int32, (16, 256), 1)
    sub16 = jax.lax.broadcasted_iota(jnp.int32, (16, 256), 0)
    target = jnp.where(lane256 < 128, 2 * p, 2 * p + 1)
    onehot2 = (sub16 == target).astype(jnp.bfloat16)
    scale_nar = jax.lax.dot_general(
        ct_ref[:, pl.ds(t * BT, BT)], onehot2, (((0,), (0,)), ((), ())),
        preferred_element_type=jnp.float32)
    s16 = scale_nar.astype(jnp.bfloat16)
    rep = D_MODEL // 128
    bcA = jnp.concatenate([s16[:, 0:128]] * rep, axis=1)
    bcB = jnp.concatenate([s16[:, 128:256]] * rep, axis=1)
    x_blk = x_ref[...]
    xs2 = jnp.concatenate([bcA * x_blk, bcB * x_blk], axis=1)

    contrib = jnp.dot(xs2, wb_ref[...], preferred_element_type=jnp.float32)

    @pl.when(p == 0)
    def _init():
        o_ref[pl.ds(t * BT, BT), :] = contrib

    @pl.when(p > 0)
    def _add():
        o_ref[pl.ds(t * BT, BT), :] += contrib


def kernel(x, spec_router_logits, shared_router_logits, spec_expert_weights, shared_expert_weights):
    x16 = x.astype(jnp.bfloat16)
    wspec4 = spec_expert_weights.reshape(NUM_SPEC // 2, 2 * D_MODEL, D_MODEL)
    wshared1 = shared_expert_weights.reshape(NUM_SHARED // 2, 2 * D_MODEL, D_MODEL)
    npair = NUM_TOTAL // 2
    return pl.pallas_call(
        _moe_body,
        grid=(npair, N_TOK // BT),
        in_specs=[
            pl.BlockSpec((NUM_SPEC, N_TOK), lambda p, t: (0, 0)),
            pl.BlockSpec((NUM_SHARED, N_TOK), lambda p, t: (0, 0)),
            pl.BlockSpec((BT, D_MODEL), lambda p, t: (t, 0)),
            pl.BlockSpec((1, 2 * D_MODEL, D_MODEL), lambda p, t: (jnp.minimum(p, NUM_SPEC // 2 - 1), 0, 0)),
            pl.BlockSpec((1, 2 * D_MODEL, D_MODEL), lambda p, t: (0, 0, 0)),
        ],
        out_specs=pl.BlockSpec((N_TOK, D_MODEL), lambda p, t: (0, 0)),
        out_shape=jax.ShapeDtypeStruct((N_TOK, D_MODEL), jnp.float32),
        scratch_shapes=[pltpu.VMEM((16, N_TOK), jnp.bfloat16),
                        pltpu.VMEM((2 * D_MODEL, D_MODEL), jnp.bfloat16)],
        compiler_params=pltpu.CompilerParams(
            dimension_semantics=("arbitrary", "arbitrary"),
        ),
    )(spec_router_logits.T, shared_router_logits.T, x16, wspec4, wshared1)
